# Initial kernel scaffold; baseline (speedup 1.0000x reference)
import jax
import jax.numpy as jnp
from jax.experimental import pallas as pl


def kernel(x, edge_index, ln_g, ln_b, W1, b1, W2, b2):
    raise NotImplementedError("write your pallas kernel here")


# speedup vs baseline: 17.7992x; 17.7992x over previous
import functools

import jax
import jax.numpy as jnp
from jax import lax
from jax.experimental import pallas as pl
from jax.experimental.pallas import tpu as pltpu
from jax.experimental.pallas import tpu_sc as plsc

N_NODES = 10000
N_EDGES = 320000
NFEAT = 128
NHID = 128
NCLASS = 40
D2 = 64

NC = 2
NS = 16
NW = NC * NS
E_PER_W = N_EDGES // NW
N_PAD = 10240
ROWS_PER_TILE = N_PAD // NS
ZCHUNK = 64

_MESH = plsc.VectorSubcoreMesh(core_axis_name="c", subcore_axis_name="s")


def _make_deg_kernel(block_e):
    n_iter = E_PER_W // block_e

    @functools.partial(
        pl.kernel,
        mesh=_MESH,
        compiler_params=pltpu.CompilerParams(use_tc_tiling_on_sc=False),
        out_type=jax.ShapeDtypeStruct((NC, N_PAD, 16), jnp.float32),
        scratch_types=[
            pltpu.VMEM((block_e,), jnp.int32),
            pltpu.VMEM((block_e, 16), jnp.float32),
            pltpu.VMEM_SHARED((N_PAD, 16), jnp.float32),
            pltpu.SemaphoreType.DMA,
        ],
    )
    def deg_kernel(dst_hbm, ones_hbm, z_hbm, out_hbm, idx_v, ones_v,
                   acc_sh, sem):
        c = lax.axis_index("c")
        s = lax.axis_index("s")
        wid = s * NC + c
        pltpu.sync_copy(ones_hbm, ones_v)
        row0 = s * ROWS_PER_TILE

        def zbody(j, carry):
            pltpu.sync_copy(z_hbm, acc_sh.at[pl.ds(row0 + j * ZCHUNK, ZCHUNK)])
            return carry

        lax.fori_loop(0, ROWS_PER_TILE // ZCHUNK, zbody, 0)
        plsc.subcore_barrier()

        def body(i, carry):
            base = wid * E_PER_W + i * block_e
            pltpu.sync_copy(dst_hbm.at[pl.ds(base, block_e)], idx_v)
            pltpu.sync_copy(ones_v, acc_sh.at[idx_v], add=True)
            return carry

        lax.fori_loop(0, n_iter, body, 0)
        plsc.subcore_barrier()
        pltpu.sync_copy(acc_sh.at[pl.ds(row0, ROWS_PER_TILE)],
                        out_hbm.at[c, pl.ds(row0, ROWS_PER_TILE)])

    return deg_kernel


def _make_spmm_kernel(d, block_e):
    n_iter = E_PER_W // block_e

    @functools.partial(
        pl.kernel,
        mesh=_MESH,
        compiler_params=pltpu.CompilerParams(use_tc_tiling_on_sc=False),
        out_type=jax.ShapeDtypeStruct((NC, N_PAD, d), jnp.float32),
        scratch_types=[
            pltpu.VMEM((block_e,), jnp.int32),
            pltpu.VMEM((block_e,), jnp.int32),
            pltpu.VMEM((block_e, d), jnp.float32),
            pltpu.VMEM_SHARED((N_PAD, d), jnp.float32),
            pltpu.SemaphoreType.DMA,
        ],
    )
    def spmm_kernel(g_hbm, src_hbm, dst_hbm, z_hbm, out_hbm, src_v, dst_v,
                    rows_v, acc_sh, sem):
        c = lax.axis_index("c")
        s = lax.axis_index("s")
        wid = s * NC + c
        row0 = s * ROWS_PER_TILE

        def zbody(j, carry):
            pltpu.sync_copy(z_hbm, acc_sh.at[pl.ds(row0 + j * ZCHUNK, ZCHUNK)])
            return carry

        lax.fori_loop(0, ROWS_PER_TILE // ZCHUNK, zbody, 0)
        plsc.subcore_barrier()

        def body(i, carry):
            base = wid * E_PER_W + i * block_e
            pltpu.sync_copy(src_hbm.at[pl.ds(base, block_e)], src_v)
            pltpu.sync_copy(dst_hbm.at[pl.ds(base, block_e)], dst_v)
            pltpu.async_copy(g_hbm.at[src_v], rows_v, sem).wait()
            pltpu.sync_copy(rows_v, acc_sh.at[dst_v], add=True)
            return carry

        lax.fori_loop(0, n_iter, body, 0)
        plsc.subcore_barrier()
        pltpu.sync_copy(acc_sh.at[pl.ds(row0, ROWS_PER_TILE)],
                        out_hbm.at[c, pl.ds(row0, ROWS_PER_TILE)])

    return spmm_kernel


_deg_kernel = _make_deg_kernel(200)
_spmm64 = _make_spmm_kernel(D2, 400)

_R = 1000


def _dis_from_degp(degp_ref):
    deg = degp_ref[0, :, 0:1] + degp_ref[1, :, 0:1] + 1.0
    return lax.rsqrt(deg)


def _tc1_body(x_ref, g_ref, b_ref, w_ref, degp_ref, outa_ref, outb_ref):
    xv = x_ref[...]
    mu = jnp.mean(xv, axis=1, keepdims=True)
    xc = xv - mu
    var = jnp.mean(xc * xc, axis=1, keepdims=True)
    h = xc * lax.rsqrt(var + 1e-5) * g_ref[...] + b_ref[...]
    dis = _dis_from_degp(degp_ref)
    g1 = jnp.dot(h, w_ref[...], preferred_element_type=jnp.float32) * dis
    outa_ref[...] = g1[:, :D2]
    outb_ref[...] = g1[:, D2:]


def _tc2_body(s1a_ref, s1b_ref, g1a_ref, g1b_ref, b1_ref, w2_ref, degp_ref,
              out_ref):
    dis = _dis_from_degp(degp_ref)
    h1a = (s1a_ref[0] + s1a_ref[1] + g1a_ref[...]) * dis
    h1b = (s1b_ref[0] + s1b_ref[1] + g1b_ref[...]) * dis
    h1 = jnp.concatenate([h1a, h1b], axis=1) + b1_ref[...]
    out_ref[...] = jnp.dot(h1, w2_ref[...],
                           preferred_element_type=jnp.float32) * dis


def _tc3_body(s2_ref, g2_ref, b2_ref, degp_ref, out_ref):
    dis = _dis_from_degp(degp_ref)
    o = (s2_ref[0] + s2_ref[1] + g2_ref[...]) * dis + b2_ref[...]
    m = jnp.max(o, axis=1, keepdims=True)
    lse = jnp.log(jnp.sum(jnp.exp(o - m), axis=1, keepdims=True)) + m
    out_ref[...] = o - lse


def _row_block(d):
    return pl.BlockSpec((_R, d), lambda i: (i, 0))


def _full(shape):
    nd = len(shape)
    return pl.BlockSpec(shape, lambda i, _nd=nd: (0,) * _nd)


def _degp_spec():
    return pl.BlockSpec((NC, _R, 16), lambda i: (0, i, 0))


def _part_spec(d):
    return pl.BlockSpec((NC, _R, d), lambda i: (0, i, 0))


def kernel(x, edge_index, ln_g, ln_b, W1, b1, W2, b2):
    src = edge_index[0].astype(jnp.int32)
    dst = edge_index[1].astype(jnp.int32)

    ones16 = jnp.ones((200, 16), jnp.float32)
    z16 = jnp.zeros((ZCHUNK, 16), jnp.float32)
    z64 = jnp.zeros((ZCHUNK, D2), jnp.float32)
    ln_g2 = ln_g.reshape(1, NFEAT)
    ln_b2 = ln_b.reshape(1, NFEAT)
    b1_2 = b1.reshape(1, NHID)
    W2p = jnp.zeros((NHID, D2), jnp.float32).at[:, :NCLASS].set(W2)
    b2p = jnp.full((1, D2), -1e30, jnp.float32).at[0, :NCLASS].set(b2)

    degp = _deg_kernel(dst, ones16, z16)

    grid = (N_NODES // _R,)
    g1a, g1b = pl.pallas_call(
        _tc1_body,
        grid=grid,
        in_specs=[_row_block(NFEAT), _full((1, NFEAT)), _full((1, NFEAT)),
                  _full((NFEAT, NHID)), _degp_spec()],
        out_specs=[_row_block(D2), _row_block(D2)],
        out_shape=[jax.ShapeDtypeStruct((N_NODES, D2), jnp.float32),
                   jax.ShapeDtypeStruct((N_NODES, D2), jnp.float32)],
    )(x, ln_g2, ln_b2, W1, degp)

    s1a = _spmm64(g1a, src, dst, z64)
    s1b = _spmm64(g1b, src, dst, z64)

    g2 = pl.pallas_call(
        _tc2_body,
        grid=grid,
        in_specs=[_part_spec(D2), _part_spec(D2), _row_block(D2),
                  _row_block(D2), _full((1, NHID)), _full((NHID, D2)),
                  _degp_spec()],
        out_specs=_row_block(D2),
        out_shape=jax.ShapeDtypeStruct((N_NODES, D2), jnp.float32),
    )(s1a, s1b, g1a, g1b, b1_2, W2p, degp)

    s2 = _spmm64(g2, src, dst, z64)

    o = pl.pallas_call(
        _tc3_body,
        grid=grid,
        in_specs=[_part_spec(D2), _row_block(D2), _full((1, D2)),
                  _degp_spec()],
        out_specs=_row_block(D2),
        out_shape=jax.ShapeDtypeStruct((N_NODES, D2), jnp.float32),
    )(s2, g2, b2p, degp)

    return o[:, :NCLASS]


# speedup vs baseline: 23.0099x; 1.2928x over previous
import functools

import jax
import jax.numpy as jnp
from jax import lax
from jax.experimental import pallas as pl
from jax.experimental.pallas import tpu as pltpu
from jax.experimental.pallas import tpu_sc as plsc

N_NODES = 10000
N_EDGES = 320000
NFEAT = 128
NHID = 128
NCLASS = 40
D2 = 64
DW = 128

NC = 2
NS = 16
NW = NC * NS
E_PER_W = N_EDGES // NW
N_PAD = 10240
ROWS_PER_TILE = N_PAD // NS
ZCHUNK = 64

_MESH = plsc.VectorSubcoreMesh(core_axis_name="c", subcore_axis_name="s")


def _make_deg_kernel(block_e):
    n_iter = E_PER_W // block_e

    @functools.partial(
        pl.kernel,
        mesh=_MESH,
        compiler_params=pltpu.CompilerParams(use_tc_tiling_on_sc=False),
        out_type=jax.ShapeDtypeStruct((NC, N_PAD, 16), jnp.float32),
        scratch_types=[
            pltpu.VMEM((n_iter, block_e), jnp.int32),
            pltpu.VMEM((block_e, 16), jnp.float32),
            pltpu.VMEM_SHARED((N_PAD, 16), jnp.float32),
            pltpu.SemaphoreType.DMA,
            pltpu.SemaphoreType.DMA,
        ],
    )
    def deg_kernel(dst_hbm, ones_hbm, z_hbm, out_hbm, idx_v, ones_v,
                   acc_sh, sem, psem):
        c = lax.axis_index("c")
        s = lax.axis_index("s")
        wid = s * NC + c
        row0 = s * ROWS_PER_TILE
        pltpu.async_copy(z_hbm, acc_sh.at[pl.ds(row0, ROWS_PER_TILE)], psem)
        pltpu.async_copy(ones_hbm, ones_v, sem)
        pltpu.async_copy(dst_hbm.at[wid], idx_v, psem)
        pltpu.make_async_copy(ones_hbm, ones_v, sem).wait()
        pltpu.make_async_copy(dst_hbm.at[wid], idx_v, psem).wait()
        pltpu.make_async_copy(z_hbm, acc_sh.at[pl.ds(row0, ROWS_PER_TILE)],
                              psem).wait()
        plsc.subcore_barrier()

        def body(i, carry):
            pltpu.async_copy(ones_v, acc_sh.at[idx_v.at[i]], sem, add=True)
            return carry

        lax.fori_loop(0, n_iter, body, 0)

        def drain(i, carry):
            pltpu.make_async_copy(ones_v, acc_sh.at[idx_v.at[i]], sem).wait()
            return carry

        lax.fori_loop(0, n_iter, drain, 0)
        plsc.subcore_barrier()
        pltpu.sync_copy(acc_sh.at[pl.ds(row0, ROWS_PER_TILE)],
                        out_hbm.at[c, pl.ds(row0, ROWS_PER_TILE)])

    return deg_kernel


def _make_spmm_kernel(d, block_e, nbuf):
    n_iter = E_PER_W // block_e
    assert n_iter % nbuf == 0

    @functools.partial(
        pl.kernel,
        mesh=_MESH,
        compiler_params=pltpu.CompilerParams(use_tc_tiling_on_sc=False),
        out_type=jax.ShapeDtypeStruct((NC, N_PAD, d), jnp.float32),
        scratch_types=[
            pltpu.VMEM((n_iter, block_e), jnp.int32),
            pltpu.VMEM((n_iter, block_e), jnp.int32),
            pltpu.VMEM((nbuf, block_e, d), jnp.float32),
            pltpu.VMEM_SHARED((N_PAD, d), jnp.float32),
        ] + [pltpu.SemaphoreType.DMA] * (2 * nbuf + 1),
    )
    def spmm_kernel(g_hbm, src_hbm, dst_hbm, z_hbm, out_hbm, src_v, dst_v,
                    rows, acc_sh, *sems):
        gsem = sems[:nbuf]
        ssem = sems[nbuf:2 * nbuf]
        psem = sems[2 * nbuf]
        c = lax.axis_index("c")
        s = lax.axis_index("s")
        wid = s * NC + c
        row0 = s * ROWS_PER_TILE

        pltpu.async_copy(z_hbm, acc_sh.at[pl.ds(row0, ROWS_PER_TILE)], psem)
        pltpu.async_copy(src_hbm.at[wid], src_v, gsem[0])
        pltpu.async_copy(dst_hbm.at[wid], dst_v, gsem[1])
        pltpu.make_async_copy(src_hbm.at[wid], src_v, gsem[0]).wait()
        pltpu.make_async_copy(dst_hbm.at[wid], dst_v, gsem[1]).wait()
        pltpu.make_async_copy(z_hbm, acc_sh.at[pl.ds(row0, ROWS_PER_TILE)],
                              psem).wait()
        plsc.subcore_barrier()

        def start_gather(i, k):
            pltpu.async_copy(g_hbm.at[src_v.at[i]], rows.at[k], gsem[k])

        def wait_gather(i, k):
            pltpu.make_async_copy(g_hbm.at[src_v.at[i]], rows.at[k],
                                  gsem[k]).wait()

        def start_scatter(i, k):
            pltpu.async_copy(rows.at[k], acc_sh.at[dst_v.at[i]], ssem[k],
                             add=True)

        def wait_scatter(i, k):
            pltpu.make_async_copy(rows.at[k], acc_sh.at[dst_v.at[i]],
                                  ssem[k]).wait()

        for k in range(nbuf):
            start_gather(k, k)

        def body(j, carry):
            i = nbuf * j
            for k in range(nbuf):
                wait_gather(i + k, k)
                start_scatter(i + k, k)
            for k in range(nbuf):
                wait_scatter(i + k, k)
                start_gather(i + nbuf + k, k)
            return carry

        lax.fori_loop(0, n_iter // nbuf - 1, body, 0)
        t = n_iter - nbuf
        for k in range(nbuf):
            wait_gather(t + k, k)
            start_scatter(t + k, k)
        for k in range(nbuf):
            wait_scatter(t + k, k)
        plsc.subcore_barrier()
        pltpu.sync_copy(acc_sh.at[pl.ds(row0, ROWS_PER_TILE)],
                        out_hbm.at[c, pl.ds(row0, ROWS_PER_TILE)])

    return spmm_kernel


_BE = 100
_deg_kernel = _make_deg_kernel(400)
_spmm128 = _make_spmm_kernel(DW, _BE, 2)

_R = 1000


def _dis_from_degp(degp_ref):
    deg = degp_ref[0, :, 0:1] + degp_ref[1, :, 0:1] + 1.0
    return lax.rsqrt(deg)


def _tc1_body(x_ref, g_ref, b_ref, w_ref, degp_ref, out_ref):
    xv = x_ref[...]
    mu = jnp.mean(xv, axis=1, keepdims=True)
    xc = xv - mu
    var = jnp.mean(xc * xc, axis=1, keepdims=True)
    h = xc * lax.rsqrt(var + 1e-5) * g_ref[...] + b_ref[...]
    dis = _dis_from_degp(degp_ref)
    out_ref[...] = jnp.dot(h, w_ref[...],
                           preferred_element_type=jnp.float32) * dis


def _tc2_body(s1_ref, g1_ref, b1_ref, w2_ref, degp_ref, out_ref):
    dis = _dis_from_degp(degp_ref)
    h1 = (s1_ref[0] + s1_ref[1] + g1_ref[...]) * dis + b1_ref[...]
    out_ref[...] = jnp.dot(h1, w2_ref[...],
                           preferred_element_type=jnp.float32) * dis


def _tc3_body(s2_ref, g2_ref, b2_ref, degp_ref, out_ref):
    dis = _dis_from_degp(degp_ref)
    o = (s2_ref[0] + s2_ref[1] + g2_ref[...]) * dis + b2_ref[...]
    m = jnp.max(o, axis=1, keepdims=True)
    lse = jnp.log(jnp.sum(jnp.exp(o - m), axis=1, keepdims=True)) + m
    out_ref[...] = (o - lse)[:, :D2]


def _row_block(d):
    return pl.BlockSpec((_R, d), lambda i: (i, 0))


def _full(shape):
    nd = len(shape)
    return pl.BlockSpec(shape, lambda i, _nd=nd: (0,) * _nd)


def _degp_spec():
    return pl.BlockSpec((NC, _R, 16), lambda i: (0, i, 0))


def _part_spec(d):
    return pl.BlockSpec((NC, _R, d), lambda i: (0, i, 0))


def kernel(x, edge_index, ln_g, ln_b, W1, b1, W2, b2):
    src = edge_index[0].astype(jnp.int32)
    dst = edge_index[1].astype(jnp.int32)

    ones16 = jnp.ones((400, 16), jnp.float32)
    z16 = jnp.zeros((ROWS_PER_TILE, 16), jnp.float32)
    z128 = jnp.zeros((ROWS_PER_TILE, DW), jnp.float32)
    ln_g2 = ln_g.reshape(1, NFEAT)
    ln_b2 = ln_b.reshape(1, NFEAT)
    b1_2 = b1.reshape(1, NHID)
    W2p = jnp.zeros((NHID, DW), jnp.float32).at[:, :NCLASS].set(W2)
    b2p = jnp.full((1, DW), -1e30, jnp.float32).at[0, :NCLASS].set(b2)

    srcd = src.reshape(NW, E_PER_W // 400, 400)
    dstd = dst.reshape(NW, E_PER_W // 400, 400)
    src3 = src.reshape(NW, E_PER_W // _BE, _BE)
    dst3 = dst.reshape(NW, E_PER_W // _BE, _BE)
    degp = _deg_kernel(dstd, ones16, z16)

    grid = (N_NODES // _R,)
    g1 = pl.pallas_call(
        _tc1_body,
        grid=grid,
        in_specs=[_row_block(NFEAT), _full((1, NFEAT)), _full((1, NFEAT)),
                  _full((NFEAT, NHID)), _degp_spec()],
        out_specs=_row_block(NHID),
        out_shape=jax.ShapeDtypeStruct((N_NODES, NHID), jnp.float32),
    )(x, ln_g2, ln_b2, W1, degp)

    s1 = _spmm128(g1, src3, dst3, z128)

    g2 = pl.pallas_call(
        _tc2_body,
        grid=grid,
        in_specs=[_part_spec(NHID), _row_block(NHID), _full((1, NHID)),
                  _full((NHID, DW)), _degp_spec()],
        out_specs=_row_block(DW),
        out_shape=jax.ShapeDtypeStruct((N_NODES, DW), jnp.float32),
    )(s1, g1, b1_2, W2p, degp)

    s2 = _spmm128(g2, src3, dst3, z128)

    o = pl.pallas_call(
        _tc3_body,
        grid=grid,
        in_specs=[_part_spec(DW), _row_block(DW), _full((1, DW)),
                  _degp_spec()],
        out_specs=_row_block(D2),
        out_shape=jax.ShapeDtypeStruct((N_NODES, D2), jnp.float32),
    )(s2, g2, b2p, degp)

    return o[:, :NCLASS]


# speedup vs baseline: 25.2106x; 1.0956x over previous
import functools

import jax
import jax.numpy as jnp
from jax import lax
from jax.experimental import pallas as pl
from jax.experimental.pallas import tpu as pltpu
from jax.experimental.pallas import tpu_sc as plsc

N_NODES = 10000
N_EDGES = 320000
NFEAT = 128
NHID = 128
NCLASS = 40
D2 = 64
DW = 128

NC = 2
NS = 16
NW = NC * NS
E_PER_W = N_EDGES // NW
N_PAD = 10240
ROWS_PER_TILE = N_PAD // NS
ZCHUNK = 64

_MESH = plsc.VectorSubcoreMesh(core_axis_name="c", subcore_axis_name="s")


def _make_deg_kernel(block_e):
    n_iter = E_PER_W // block_e

    @functools.partial(
        pl.kernel,
        mesh=_MESH,
        compiler_params=pltpu.CompilerParams(use_tc_tiling_on_sc=False),
        out_type=jax.ShapeDtypeStruct((NC, N_PAD, 16), jnp.float32),
        scratch_types=[
            pltpu.VMEM((n_iter, block_e), jnp.int32),
            pltpu.VMEM((block_e, 16), jnp.float32),
            pltpu.VMEM_SHARED((N_PAD, 16), jnp.float32),
            pltpu.SemaphoreType.DMA,
            pltpu.SemaphoreType.DMA,
        ],
    )
    def deg_kernel(dst_hbm, ones_hbm, z_hbm, out_hbm, idx_v, ones_v,
                   acc_sh, sem, psem):
        c = lax.axis_index("c")
        s = lax.axis_index("s")
        wid = s * NC + c
        row0 = s * ROWS_PER_TILE
        pltpu.async_copy(z_hbm, acc_sh.at[pl.ds(row0, ROWS_PER_TILE)], psem)
        pltpu.async_copy(ones_hbm, ones_v, sem)
        pltpu.async_copy(dst_hbm.at[wid], idx_v, psem)
        pltpu.make_async_copy(ones_hbm, ones_v, sem).wait()
        pltpu.make_async_copy(dst_hbm.at[wid], idx_v, psem).wait()
        pltpu.make_async_copy(z_hbm, acc_sh.at[pl.ds(row0, ROWS_PER_TILE)],
                              psem).wait()
        plsc.subcore_barrier()

        def body(i, carry):
            pltpu.async_copy(ones_v, acc_sh.at[idx_v.at[i]], sem, add=True)
            return carry

        lax.fori_loop(0, n_iter, body, 0)

        def drain(i, carry):
            pltpu.make_async_copy(ones_v, acc_sh.at[idx_v.at[i]], sem).wait()
            return carry

        lax.fori_loop(0, n_iter, drain, 0)
        plsc.subcore_barrier()
        pltpu.sync_copy(acc_sh.at[pl.ds(row0, ROWS_PER_TILE)],
                        out_hbm.at[c, pl.ds(row0, ROWS_PER_TILE)])

    return deg_kernel


def _make_spmm_kernel(d, block_e, nbuf):
    n_iter = E_PER_W // block_e
    assert n_iter % nbuf == 0

    @functools.partial(
        pl.kernel,
        mesh=_MESH,
        compiler_params=pltpu.CompilerParams(use_tc_tiling_on_sc=False),
        out_type=jax.ShapeDtypeStruct((NC, N_PAD, d), jnp.float32),
        scratch_types=[
            pltpu.VMEM((n_iter, block_e), jnp.int32),
            pltpu.VMEM((n_iter, block_e), jnp.int32),
            pltpu.VMEM((nbuf, block_e, d), jnp.float32),
            pltpu.VMEM_SHARED((N_PAD, d), jnp.float32),
        ] + [pltpu.SemaphoreType.DMA] * (2 * nbuf + 1),
    )
    def spmm_kernel(g_hbm, src_hbm, dst_hbm, z_hbm, out_hbm, src_v, dst_v,
                    rows, acc_sh, *sems):
        gsem = sems[:nbuf]
        ssem = sems[nbuf:2 * nbuf]
        psem = sems[2 * nbuf]
        c = lax.axis_index("c")
        s = lax.axis_index("s")
        wid = s * NC + c
        row0 = s * ROWS_PER_TILE

        pltpu.async_copy(z_hbm, acc_sh.at[pl.ds(row0, ROWS_PER_TILE)], psem)
        pltpu.async_copy(src_hbm.at[wid], src_v, gsem[0])
        pltpu.async_copy(dst_hbm.at[wid], dst_v, gsem[1])
        pltpu.make_async_copy(src_hbm.at[wid], src_v, gsem[0]).wait()
        pltpu.make_async_copy(dst_hbm.at[wid], dst_v, gsem[1]).wait()
        pltpu.make_async_copy(z_hbm, acc_sh.at[pl.ds(row0, ROWS_PER_TILE)],
                              psem).wait()
        plsc.subcore_barrier()

        def start_gather(i, k):
            pltpu.async_copy(g_hbm.at[src_v.at[i]], rows.at[k], gsem[k])

        def wait_gather(i, k):
            pltpu.make_async_copy(g_hbm.at[src_v.at[i]], rows.at[k],
                                  gsem[k]).wait()

        def start_scatter(i, k):
            pltpu.async_copy(rows.at[k], acc_sh.at[dst_v.at[i]], ssem[k],
                             add=True)

        def wait_scatter(i, k):
            pltpu.make_async_copy(rows.at[k], acc_sh.at[dst_v.at[i]],
                                  ssem[k]).wait()

        for k in range(nbuf):
            start_gather(k, k)

        def body(j, carry):
            i = nbuf * j
            for k in range(nbuf):
                wait_gather(i + k, k)
                start_scatter(i + k, k)
            for k in range(nbuf):
                wait_scatter(i + k, k)
                start_gather(i + nbuf + k, k)
            return carry

        lax.fori_loop(0, n_iter // nbuf - 1, body, 0)
        t = n_iter - nbuf
        for k in range(nbuf):
            wait_gather(t + k, k)
            start_scatter(t + k, k)
        for k in range(nbuf):
            wait_scatter(t + k, k)
        plsc.subcore_barrier()
        pltpu.sync_copy(acc_sh.at[pl.ds(row0, ROWS_PER_TILE)],
                        out_hbm.at[c, pl.ds(row0, ROWS_PER_TILE)])

    return spmm_kernel


_BE = 500
_deg_kernel = _make_deg_kernel(400)
_spmm64 = _make_spmm_kernel(D2, _BE, 2)

_R = 1000


def _dis_from_degp(degp_ref):
    deg = degp_ref[0, :, 0:1] + degp_ref[1, :, 0:1] + 1.0
    return lax.rsqrt(deg)


def _tc1_body(x_ref, g_ref, b_ref, w_ref, degp_ref, outa_ref, outb_ref):
    xv = x_ref[...]
    mu = jnp.mean(xv, axis=1, keepdims=True)
    xc = xv - mu
    var = jnp.mean(xc * xc, axis=1, keepdims=True)
    h = xc * lax.rsqrt(var + 1e-5) * g_ref[...] + b_ref[...]
    dis = _dis_from_degp(degp_ref)
    g1 = jnp.dot(h, w_ref[...], preferred_element_type=jnp.float32) * dis
    outa_ref[...] = g1[:, :D2]
    outb_ref[...] = g1[:, D2:]


def _tc2_body(s1a_ref, s1b_ref, g1a_ref, g1b_ref, b1_ref, w2_ref, degp_ref,
              out_ref):
    dis = _dis_from_degp(degp_ref)
    h1a = (s1a_ref[0] + s1a_ref[1] + g1a_ref[...]) * dis
    h1b = (s1b_ref[0] + s1b_ref[1] + g1b_ref[...]) * dis
    h1 = jnp.concatenate([h1a, h1b], axis=1) + b1_ref[...]
    out_ref[...] = jnp.dot(h1, w2_ref[...],
                           preferred_element_type=jnp.float32) * dis


def _tc3_body(s2_ref, g2_ref, b2_ref, degp_ref, out_ref):
    dis = _dis_from_degp(degp_ref)
    o = (s2_ref[0] + s2_ref[1] + g2_ref[...]) * dis + b2_ref[...]
    m = jnp.max(o, axis=1, keepdims=True)
    lse = jnp.log(jnp.sum(jnp.exp(o - m), axis=1, keepdims=True)) + m
    out_ref[...] = o - lse


def _row_block(d):
    return pl.BlockSpec((_R, d), lambda i: (i, 0))


def _full(shape):
    nd = len(shape)
    return pl.BlockSpec(shape, lambda i, _nd=nd: (0,) * _nd)


def _degp_spec():
    return pl.BlockSpec((NC, _R, 16), lambda i: (0, i, 0))


def _part_spec(d):
    return pl.BlockSpec((NC, _R, d), lambda i: (0, i, 0))


def kernel(x, edge_index, ln_g, ln_b, W1, b1, W2, b2):
    src = edge_index[0].astype(jnp.int32)
    dst = edge_index[1].astype(jnp.int32)

    ones16 = jnp.ones((400, 16), jnp.float32)
    z16 = jnp.zeros((ROWS_PER_TILE, 16), jnp.float32)
    z64 = jnp.zeros((ROWS_PER_TILE, D2), jnp.float32)
    ln_g2 = ln_g.reshape(1, NFEAT)
    ln_b2 = ln_b.reshape(1, NFEAT)
    b1_2 = b1.reshape(1, NHID)
    W2p = jnp.zeros((NHID, D2), jnp.float32).at[:, :NCLASS].set(W2)
    b2p = jnp.full((1, D2), -1e30, jnp.float32).at[0, :NCLASS].set(b2)

    srcd = src.reshape(NW, E_PER_W // 400, 400)
    dstd = dst.reshape(NW, E_PER_W // 400, 400)
    src3 = src.reshape(NW, E_PER_W // _BE, _BE)
    dst3 = dst.reshape(NW, E_PER_W // _BE, _BE)
    degp = _deg_kernel(dstd, ones16, z16)

    grid = (N_NODES // _R,)
    g1a, g1b = pl.pallas_call(
        _tc1_body,
        grid=grid,
        in_specs=[_row_block(NFEAT), _full((1, NFEAT)), _full((1, NFEAT)),
                  _full((NFEAT, NHID)), _degp_spec()],
        out_specs=[_row_block(D2), _row_block(D2)],
        out_shape=[jax.ShapeDtypeStruct((N_NODES, D2), jnp.float32),
                   jax.ShapeDtypeStruct((N_NODES, D2), jnp.float32)],
    )(x, ln_g2, ln_b2, W1, degp)

    s1a = _spmm64(g1a, src3, dst3, z64)
    s1b = _spmm64(g1b, src3, dst3, z64)

    g2 = pl.pallas_call(
        _tc2_body,
        grid=grid,
        in_specs=[_part_spec(D2), _part_spec(D2), _row_block(D2),
                  _row_block(D2), _full((1, NHID)), _full((NHID, D2)),
                  _degp_spec()],
        out_specs=_row_block(D2),
        out_shape=jax.ShapeDtypeStruct((N_NODES, D2), jnp.float32),
    )(s1a, s1b, g1a, g1b, b1_2, W2p, degp)

    s2 = _spmm64(g2, src3, dst3, z64)

    o = pl.pallas_call(
        _tc3_body,
        grid=grid,
        in_specs=[_part_spec(D2), _row_block(D2), _full((1, D2)),
                  _degp_spec()],
        out_specs=_row_block(D2),
        out_shape=jax.ShapeDtypeStruct((N_NODES, D2), jnp.float32),
    )(s2, g2, b2p, degp)

    return o[:, :NCLASS]


# speedup vs baseline: 28.7187x; 1.1391x over previous
import functools

import jax
import jax.numpy as jnp
from jax import lax
from jax.experimental import pallas as pl
from jax.experimental.pallas import tpu as pltpu
from jax.experimental.pallas import tpu_sc as plsc

N_NODES = 10000
N_EDGES = 320000
NFEAT = 128
NHID = 128
NCLASS = 40
D2 = 64
DW = 128

NC = 2
NS = 16
NW = NC * NS
E_PER_W = N_EDGES // NW
N_PAD = 10240
ROWS_PER_TILE = N_PAD // NS
ZCHUNK = 64

_MESH = plsc.VectorSubcoreMesh(core_axis_name="c", subcore_axis_name="s")


def _make_deg_kernel(block_e):
    n_iter = E_PER_W // block_e

    @functools.partial(
        pl.kernel,
        mesh=_MESH,
        compiler_params=pltpu.CompilerParams(use_tc_tiling_on_sc=False),
        out_type=jax.ShapeDtypeStruct((NC, N_PAD, 16), jnp.float32),
        scratch_types=[
            pltpu.VMEM((n_iter, block_e), jnp.int32),
            pltpu.VMEM((block_e, 16), jnp.float32),
            pltpu.VMEM_SHARED((N_PAD, 16), jnp.float32),
            pltpu.SemaphoreType.DMA,
            pltpu.SemaphoreType.DMA,
        ],
    )
    def deg_kernel(dst_hbm, ones_hbm, z_hbm, out_hbm, idx_v, ones_v,
                   acc_sh, sem, psem):
        c = lax.axis_index("c")
        s = lax.axis_index("s")
        wid = s * NC + c
        row0 = s * ROWS_PER_TILE
        pltpu.async_copy(z_hbm, acc_sh.at[pl.ds(row0, ROWS_PER_TILE)], psem)
        pltpu.async_copy(ones_hbm, ones_v, sem)
        pltpu.async_copy(dst_hbm.at[wid], idx_v, psem)
        pltpu.make_async_copy(ones_hbm, ones_v, sem).wait()
        pltpu.make_async_copy(dst_hbm.at[wid], idx_v, psem).wait()
        pltpu.make_async_copy(z_hbm, acc_sh.at[pl.ds(row0, ROWS_PER_TILE)],
                              psem).wait()
        plsc.subcore_barrier()

        def body(i, carry):
            pltpu.async_copy(ones_v, acc_sh.at[idx_v.at[i]], sem, add=True)
            return carry

        lax.fori_loop(0, n_iter, body, 0)

        def drain(i, carry):
            pltpu.make_async_copy(ones_v, acc_sh.at[idx_v.at[i]], sem).wait()
            return carry

        lax.fori_loop(0, n_iter, drain, 0)
        plsc.subcore_barrier()
        pltpu.sync_copy(acc_sh.at[pl.ds(row0, ROWS_PER_TILE)],
                        out_hbm.at[c, pl.ds(row0, ROWS_PER_TILE)])

    return deg_kernel


def _make_spmm_kernel(d, block_e, nbuf):
    n_iter = E_PER_W // block_e
    assert n_iter % nbuf == 0

    @functools.partial(
        pl.kernel,
        mesh=_MESH,
        compiler_params=pltpu.CompilerParams(use_tc_tiling_on_sc=False),
        out_type=jax.ShapeDtypeStruct((NC, N_PAD, d), jnp.float32),
        scratch_types=[
            pltpu.VMEM((n_iter, block_e), jnp.int32),
            pltpu.VMEM((n_iter, block_e), jnp.int32),
            pltpu.VMEM((nbuf, block_e, d), jnp.float32),
            pltpu.VMEM_SHARED((N_PAD, d), jnp.float32),
        ] + [pltpu.SemaphoreType.DMA] * (2 * nbuf + 1),
    )
    def spmm_kernel(g_hbm, src_hbm, dst_hbm, z_hbm, out_hbm, src_v, dst_v,
                    rows, acc_sh, *sems):
        gsem = sems[:nbuf]
        ssem = sems[nbuf:2 * nbuf]
        psem = sems[2 * nbuf]
        c = lax.axis_index("c")
        s = lax.axis_index("s")
        wid = s * NC + c
        row0 = s * ROWS_PER_TILE

        pltpu.async_copy(z_hbm, acc_sh.at[pl.ds(row0, ROWS_PER_TILE)], psem)
        pltpu.async_copy(src_hbm.at[wid], src_v, gsem[0])
        pltpu.async_copy(dst_hbm.at[wid], dst_v, gsem[1])
        pltpu.make_async_copy(src_hbm.at[wid], src_v, gsem[0]).wait()
        pltpu.make_async_copy(dst_hbm.at[wid], dst_v, gsem[1]).wait()
        pltpu.make_async_copy(z_hbm, acc_sh.at[pl.ds(row0, ROWS_PER_TILE)],
                              psem).wait()
        plsc.subcore_barrier()

        def start_gather(i, k):
            pltpu.async_copy(g_hbm.at[src_v.at[i]], rows.at[k], gsem[k])

        def wait_gather(i, k):
            pltpu.make_async_copy(g_hbm.at[src_v.at[i]], rows.at[k],
                                  gsem[k]).wait()

        def start_scatter(i, k):
            pltpu.async_copy(rows.at[k], acc_sh.at[dst_v.at[i]], ssem[k],
                             add=True)

        def wait_scatter(i, k):
            pltpu.make_async_copy(rows.at[k], acc_sh.at[dst_v.at[i]],
                                  ssem[k]).wait()

        for k in range(nbuf):
            start_gather(k, k)

        def body(j, carry):
            i = nbuf * j
            for k in range(nbuf):
                wait_gather(i + k, k)
                start_scatter(i + k, k)
            for k in range(nbuf):
                wait_scatter(i + k, k)
                start_gather(i + nbuf + k, k)
            return carry

        lax.fori_loop(0, n_iter // nbuf - 1, body, 0)
        t = n_iter - nbuf
        for k in range(nbuf):
            wait_gather(t + k, k)
            start_scatter(t + k, k)
        for k in range(nbuf):
            wait_scatter(t + k, k)
        plsc.subcore_barrier()
        pltpu.sync_copy(acc_sh.at[pl.ds(row0, ROWS_PER_TILE)],
                        out_hbm.at[c, pl.ds(row0, ROWS_PER_TILE)])

    return spmm_kernel


_BE = 200
_deg_kernel = _make_deg_kernel(400)
_spmm64 = _make_spmm_kernel(D2, _BE, 5)

_R = 1000


def _dis_from_degp(degp_ref):
    deg = degp_ref[0, :, 0:1] + degp_ref[1, :, 0:1] + 1.0
    return lax.rsqrt(deg)


def _tc1_body(x_ref, g_ref, b_ref, w_ref, degp_ref, outa_ref, outb_ref):
    xv = x_ref[...]
    mu = jnp.mean(xv, axis=1, keepdims=True)
    xc = xv - mu
    var = jnp.mean(xc * xc, axis=1, keepdims=True)
    h = xc * lax.rsqrt(var + 1e-5) * g_ref[...] + b_ref[...]
    dis = _dis_from_degp(degp_ref)
    g1 = jnp.dot(h, w_ref[...], preferred_element_type=jnp.float32) * dis
    outa_ref[...] = g1[:, :D2]
    outb_ref[...] = g1[:, D2:]


def _tc2_body(s1a_ref, s1b_ref, g1a_ref, g1b_ref, b1_ref, w2_ref, degp_ref,
              out_ref):
    dis = _dis_from_degp(degp_ref)
    h1a = (s1a_ref[0] + s1a_ref[1] + g1a_ref[...]) * dis
    h1b = (s1b_ref[0] + s1b_ref[1] + g1b_ref[...]) * dis
    h1 = jnp.concatenate([h1a, h1b], axis=1) + b1_ref[...]
    out_ref[...] = jnp.dot(h1, w2_ref[...],
                           preferred_element_type=jnp.float32) * dis


def _tc3_body(s2_ref, g2_ref, b2_ref, degp_ref, out_ref):
    dis = _dis_from_degp(degp_ref)
    o = (s2_ref[0] + s2_ref[1] + g2_ref[...]) * dis + b2_ref[...]
    m = jnp.max(o, axis=1, keepdims=True)
    lse = jnp.log(jnp.sum(jnp.exp(o - m), axis=1, keepdims=True)) + m
    out_ref[...] = o - lse


def _row_block(d):
    return pl.BlockSpec((_R, d), lambda i: (i, 0))


def _full(shape):
    nd = len(shape)
    return pl.BlockSpec(shape, lambda i, _nd=nd: (0,) * _nd)


def _degp_spec():
    return pl.BlockSpec((NC, _R, 16), lambda i: (0, i, 0))


def _part_spec(d):
    return pl.BlockSpec((NC, _R, d), lambda i: (0, i, 0))


def kernel(x, edge_index, ln_g, ln_b, W1, b1, W2, b2):
    src = edge_index[0].astype(jnp.int32)
    dst = edge_index[1].astype(jnp.int32)

    ones16 = jnp.ones((400, 16), jnp.float32)
    z16 = jnp.zeros((ROWS_PER_TILE, 16), jnp.float32)
    z64 = jnp.zeros((ROWS_PER_TILE, D2), jnp.float32)
    ln_g2 = ln_g.reshape(1, NFEAT)
    ln_b2 = ln_b.reshape(1, NFEAT)
    b1_2 = b1.reshape(1, NHID)
    W2p = jnp.zeros((NHID, D2), jnp.float32).at[:, :NCLASS].set(W2)
    b2p = jnp.full((1, D2), -1e30, jnp.float32).at[0, :NCLASS].set(b2)

    srcd = src.reshape(NW, E_PER_W // 400, 400)
    dstd = dst.reshape(NW, E_PER_W // 400, 400)
    src3 = src.reshape(NW, E_PER_W // _BE, _BE)
    dst3 = dst.reshape(NW, E_PER_W // _BE, _BE)
    degp = _deg_kernel(dstd, ones16, z16)

    grid = (N_NODES // _R,)
    g1a, g1b = pl.pallas_call(
        _tc1_body,
        grid=grid,
        in_specs=[_row_block(NFEAT), _full((1, NFEAT)), _full((1, NFEAT)),
                  _full((NFEAT, NHID)), _degp_spec()],
        out_specs=[_row_block(D2), _row_block(D2)],
        out_shape=[jax.ShapeDtypeStruct((N_NODES, D2), jnp.float32),
                   jax.ShapeDtypeStruct((N_NODES, D2), jnp.float32)],
    )(x, ln_g2, ln_b2, W1, degp)

    s1a = _spmm64(g1a, src3, dst3, z64)
    s1b = _spmm64(g1b, src3, dst3, z64)

    g2 = pl.pallas_call(
        _tc2_body,
        grid=grid,
        in_specs=[_part_spec(D2), _part_spec(D2), _row_block(D2),
                  _row_block(D2), _full((1, NHID)), _full((NHID, D2)),
                  _degp_spec()],
        out_specs=_row_block(D2),
        out_shape=jax.ShapeDtypeStruct((N_NODES, D2), jnp.float32),
    )(s1a, s1b, g1a, g1b, b1_2, W2p, degp)

    s2 = _spmm64(g2, src3, dst3, z64)

    o = pl.pallas_call(
        _tc3_body,
        grid=grid,
        in_specs=[_part_spec(D2), _row_block(D2), _full((1, D2)),
                  _degp_spec()],
        out_specs=_row_block(D2),
        out_shape=jax.ShapeDtypeStruct((N_NODES, D2), jnp.float32),
    )(s2, g2, b2p, degp)

    return o[:, :NCLASS]


# speedup vs baseline: 29.3521x; 1.0221x over previous
import functools

import jax
import jax.numpy as jnp
from jax import lax
from jax.experimental import pallas as pl
from jax.experimental.pallas import tpu as pltpu
from jax.experimental.pallas import tpu_sc as plsc

N_NODES = 10000
N_EDGES = 320000
NFEAT = 128
NHID = 128
NCLASS = 40
D2 = 64
DW = 128

NC = 2
NS = 16
NW = NC * NS
E_PER_W = N_EDGES // NW
N_PAD = 10240
ROWS_PER_TILE = N_PAD // NS
ZCHUNK = 64

_MESH = plsc.VectorSubcoreMesh(core_axis_name="c", subcore_axis_name="s")


def _make_deg_kernel(block_e):
    n_iter = E_PER_W // block_e

    @functools.partial(
        pl.kernel,
        mesh=_MESH,
        compiler_params=pltpu.CompilerParams(use_tc_tiling_on_sc=False),
        out_type=jax.ShapeDtypeStruct((NC, N_PAD, 16), jnp.float32),
        scratch_types=[
            pltpu.VMEM((n_iter, block_e), jnp.int32),
            pltpu.VMEM((block_e, 16), jnp.float32),
            pltpu.VMEM_SHARED((N_PAD, 16), jnp.float32),
            pltpu.SemaphoreType.DMA,
            pltpu.SemaphoreType.DMA,
        ],
    )
    def deg_kernel(dst_hbm, ones_hbm, z_hbm, out_hbm, idx_v, ones_v,
                   acc_sh, sem, psem):
        c = lax.axis_index("c")
        s = lax.axis_index("s")
        wid = s * NC + c
        row0 = s * ROWS_PER_TILE
        pltpu.async_copy(z_hbm, acc_sh.at[pl.ds(row0, ROWS_PER_TILE)], psem)
        pltpu.async_copy(ones_hbm, ones_v, sem)
        pltpu.async_copy(dst_hbm.at[wid], idx_v, psem)
        pltpu.make_async_copy(ones_hbm, ones_v, sem).wait()
        pltpu.make_async_copy(dst_hbm.at[wid], idx_v, psem).wait()
        pltpu.make_async_copy(z_hbm, acc_sh.at[pl.ds(row0, ROWS_PER_TILE)],
                              psem).wait()
        plsc.subcore_barrier()

        def body(i, carry):
            pltpu.async_copy(ones_v, acc_sh.at[idx_v.at[i]], sem, add=True)
            return carry

        lax.fori_loop(0, n_iter, body, 0)

        def drain(i, carry):
            pltpu.make_async_copy(ones_v, acc_sh.at[idx_v.at[i]], sem).wait()
            return carry

        lax.fori_loop(0, n_iter, drain, 0)
        plsc.subcore_barrier()
        pltpu.sync_copy(acc_sh.at[pl.ds(row0, ROWS_PER_TILE)],
                        out_hbm.at[c, pl.ds(row0, ROWS_PER_TILE)])

    return deg_kernel


def _make_spmm_kernel(d, block_e, nbuf):
    n_iter = E_PER_W // block_e
    assert n_iter % nbuf == 0

    @functools.partial(
        pl.kernel,
        mesh=_MESH,
        compiler_params=pltpu.CompilerParams(use_tc_tiling_on_sc=False),
        out_type=jax.ShapeDtypeStruct((NC, N_PAD, d), jnp.float32),
        scratch_types=[
            pltpu.VMEM((n_iter, block_e), jnp.int32),
            pltpu.VMEM((n_iter, block_e), jnp.int32),
            pltpu.VMEM((nbuf, block_e, d), jnp.float32),
            pltpu.VMEM_SHARED((N_PAD, d), jnp.float32),
        ] + [pltpu.SemaphoreType.DMA] * (2 * nbuf + 1),
    )
    def spmm_kernel(g_hbm, src_hbm, dst_hbm, z_hbm, out_hbm, src_v, dst_v,
                    rows, acc_sh, *sems):
        gsem = sems[:nbuf]
        ssem = sems[nbuf:2 * nbuf]
        psem = sems[2 * nbuf]
        c = lax.axis_index("c")
        s = lax.axis_index("s")
        wid = s * NC + c
        row0 = s * ROWS_PER_TILE

        pltpu.async_copy(z_hbm, acc_sh.at[pl.ds(row0, ROWS_PER_TILE)], psem)
        pltpu.async_copy(src_hbm.at[wid], src_v, gsem[0])
        pltpu.async_copy(dst_hbm.at[wid], dst_v, gsem[1])
        pltpu.make_async_copy(src_hbm.at[wid], src_v, gsem[0]).wait()
        pltpu.make_async_copy(dst_hbm.at[wid], dst_v, gsem[1]).wait()
        pltpu.make_async_copy(z_hbm, acc_sh.at[pl.ds(row0, ROWS_PER_TILE)],
                              psem).wait()
        plsc.subcore_barrier()

        def start_gather(i, k):
            pltpu.async_copy(g_hbm.at[src_v.at[i]], rows.at[k], gsem[k])

        def wait_gather(i, k):
            pltpu.make_async_copy(g_hbm.at[src_v.at[i]], rows.at[k],
                                  gsem[k]).wait()

        def start_scatter(i, k):
            pltpu.async_copy(rows.at[k], acc_sh.at[dst_v.at[i]], ssem[k],
                             add=True)

        def wait_scatter(i, k):
            pltpu.make_async_copy(rows.at[k], acc_sh.at[dst_v.at[i]],
                                  ssem[k]).wait()

        for k in range(nbuf):
            start_gather(k, k)

        def body(j, carry):
            i = nbuf * j
            for k in range(nbuf):
                wait_gather(i + k, k)
                start_scatter(i + k, k)
            for k in range(nbuf):
                wait_scatter(i + k, k)
                start_gather(i + nbuf + k, k)
            return carry

        lax.fori_loop(0, n_iter // nbuf - 1, body, 0)
        t = n_iter - nbuf
        for k in range(nbuf):
            wait_gather(t + k, k)
            start_scatter(t + k, k)
        for k in range(nbuf):
            wait_scatter(t + k, k)
        plsc.subcore_barrier()
        pltpu.sync_copy(acc_sh.at[pl.ds(row0, ROWS_PER_TILE)],
                        out_hbm.at[c, pl.ds(row0, ROWS_PER_TILE)])

    return spmm_kernel


_BE = 100
_deg_kernel = _make_deg_kernel(400)
_spmm64 = _make_spmm_kernel(D2, _BE, 10)

_R = 1000


def _dis_from_degp(degp_ref):
    deg = degp_ref[0, :, 0:1] + degp_ref[1, :, 0:1] + 1.0
    return lax.rsqrt(deg)


def _tc1_body(x_ref, g_ref, b_ref, w_ref, degp_ref, outa_ref, outb_ref):
    xv = x_ref[...]
    mu = jnp.mean(xv, axis=1, keepdims=True)
    xc = xv - mu
    var = jnp.mean(xc * xc, axis=1, keepdims=True)
    h = xc * lax.rsqrt(var + 1e-5) * g_ref[...] + b_ref[...]
    dis = _dis_from_degp(degp_ref)
    g1 = jnp.dot(h, w_ref[...], preferred_element_type=jnp.float32) * dis
    outa_ref[...] = g1[:, :D2]
    outb_ref[...] = g1[:, D2:]


def _tc2_body(s1a_ref, s1b_ref, g1a_ref, g1b_ref, b1_ref, w2_ref, degp_ref,
              out_ref):
    dis = _dis_from_degp(degp_ref)
    h1a = (s1a_ref[0] + s1a_ref[1] + g1a_ref[...]) * dis
    h1b = (s1b_ref[0] + s1b_ref[1] + g1b_ref[...]) * dis
    h1 = jnp.concatenate([h1a, h1b], axis=1) + b1_ref[...]
    out_ref[...] = jnp.dot(h1, w2_ref[...],
                           preferred_element_type=jnp.float32) * dis


def _tc3_body(s2_ref, g2_ref, b2_ref, degp_ref, out_ref):
    dis = _dis_from_degp(degp_ref)
    o = (s2_ref[0] + s2_ref[1] + g2_ref[...]) * dis + b2_ref[...]
    m = jnp.max(o, axis=1, keepdims=True)
    lse = jnp.log(jnp.sum(jnp.exp(o - m), axis=1, keepdims=True)) + m
    out_ref[...] = o - lse


def _row_block(d):
    return pl.BlockSpec((_R, d), lambda i: (i, 0))


def _full(shape):
    nd = len(shape)
    return pl.BlockSpec(shape, lambda i, _nd=nd: (0,) * _nd)


def _degp_spec():
    return pl.BlockSpec((NC, _R, 16), lambda i: (0, i, 0))


def _part_spec(d):
    return pl.BlockSpec((NC, _R, d), lambda i: (0, i, 0))


def kernel(x, edge_index, ln_g, ln_b, W1, b1, W2, b2):
    src = edge_index[0].astype(jnp.int32)
    dst = edge_index[1].astype(jnp.int32)

    ones16 = jnp.ones((400, 16), jnp.float32)
    z16 = jnp.zeros((ROWS_PER_TILE, 16), jnp.float32)
    z64 = jnp.zeros((ROWS_PER_TILE, D2), jnp.float32)
    ln_g2 = ln_g.reshape(1, NFEAT)
    ln_b2 = ln_b.reshape(1, NFEAT)
    b1_2 = b1.reshape(1, NHID)
    W2p = jnp.zeros((NHID, D2), jnp.float32).at[:, :NCLASS].set(W2)
    b2p = jnp.full((1, D2), -1e30, jnp.float32).at[0, :NCLASS].set(b2)

    srcd = src.reshape(NW, E_PER_W // 400, 400)
    dstd = dst.reshape(NW, E_PER_W // 400, 400)
    src3 = src.reshape(NW, E_PER_W // _BE, _BE)
    dst3 = dst.reshape(NW, E_PER_W // _BE, _BE)
    degp = _deg_kernel(dstd, ones16, z16)

    grid = (N_NODES // _R,)
    g1a, g1b = pl.pallas_call(
        _tc1_body,
        grid=grid,
        in_specs=[_row_block(NFEAT), _full((1, NFEAT)), _full((1, NFEAT)),
                  _full((NFEAT, NHID)), _degp_spec()],
        out_specs=[_row_block(D2), _row_block(D2)],
        out_shape=[jax.ShapeDtypeStruct((N_NODES, D2), jnp.float32),
                   jax.ShapeDtypeStruct((N_NODES, D2), jnp.float32)],
    )(x, ln_g2, ln_b2, W1, degp)

    s1a = _spmm64(g1a, src3, dst3, z64)
    s1b = _spmm64(g1b, src3, dst3, z64)

    g2 = pl.pallas_call(
        _tc2_body,
        grid=grid,
        in_specs=[_part_spec(D2), _part_spec(D2), _row_block(D2),
                  _row_block(D2), _full((1, NHID)), _full((NHID, D2)),
                  _degp_spec()],
        out_specs=_row_block(D2),
        out_shape=jax.ShapeDtypeStruct((N_NODES, D2), jnp.float32),
    )(s1a, s1b, g1a, g1b, b1_2, W2p, degp)

    s2 = _spmm64(g2, src3, dst3, z64)

    o = pl.pallas_call(
        _tc3_body,
        grid=grid,
        in_specs=[_part_spec(D2), _row_block(D2), _full((1, D2)),
                  _degp_spec()],
        out_specs=_row_block(D2),
        out_shape=jax.ShapeDtypeStruct((N_NODES, D2), jnp.float32),
    )(s2, g2, b2p, degp)

    return o[:, :NCLASS]
